# baseline (speedup 1.0000x reference)
import jax
import jax.numpy as jnp
from jax import lax
from jax.experimental import pallas as pl
from jax.experimental.pallas import tpu as pltpu
from jax.experimental.pallas import tpu_sc as plsc

B, C, H, W = 16, 64, 128, 128
HW = H * W
CHW = C * HW
N = 8192
NC, NS = 2, 16
NW = NC * NS
NPW = N // NW
EPW = NPW * C
ROWS = EPW // 128
SCALE = 1.0 / (N + 0.0001)


NCHUNK = 16
GPC = (NPW // 16) // NCHUNK
RPC = ROWS // NCHUNK
EPC = EPW // NCHUNK


def _body(out_hbm, tgt_hbm, ind_hbm, bat_hbm, res_hbm,
          indv, batv, idx2, pred2, tgtv, accv, finv, shared,
          sem_t, sem_g0, sem_g1, sem_g2, sem_g3,
          sem_g4, sem_g5, sem_g6, sem_g7,
          sem_g8, sem_g9, sem_ga, sem_gb,
          sem_gc, sem_gd, sem_ge, sem_gf):
    cid = lax.axis_index("c")
    sid = lax.axis_index("s")
    wid = cid * NS + sid
    n0 = wid * NPW
    sems = (sem_g0, sem_g1, sem_g2, sem_g3, sem_g4, sem_g5, sem_g6, sem_g7,
            sem_g8, sem_g9, sem_ga, sem_gb, sem_gc, sem_gd, sem_ge, sem_gf)

    tcopy = pltpu.async_copy(tgt_hbm.at[pl.ds(n0 * C, EPW)], tgtv, sem_t)
    icopy = pltpu.async_copy(ind_hbm.at[pl.ds(n0, NPW)], indv, sem_g0)
    bcopy = pltpu.async_copy(bat_hbm.at[pl.ds(n0, NPW)], batv, sem_g0)
    icopy.wait()
    bcopy.wait()

    iota = lax.iota(jnp.int32, 16)

    def build(g, carry):
        ind16 = indv[pl.ds(g * 16, 16)]
        bat16 = batv[pl.ds(g * 16, 16)]
        base16 = bat16 * CHW + ind16
        p0 = (g * 16 + iota) * C
        for c in range(C):
            plsc.store_scatter(idx2, [p0 + c], base16 + c * HW)
        return carry

    for k in range(NCHUNK):
        lax.fori_loop(k * GPC, (k + 1) * GPC, build, 0, unroll=False)
        pltpu.async_copy(out_hbm.at[idx2.at[pl.ds(k * EPC, EPC)]],
                         pred2.at[pl.ds(k * EPC, EPC)], sems[k])

    tcopy.wait()

    zero = jnp.zeros((16,), jnp.float32)

    def comp(j, accs):
        a0, a1, a2, a3 = accs
        base = j * 128
        d = [None] * 8
        for q in range(8):
            pv = pred2[pl.ds(base + q * 16, 16)]
            tv = tgtv[pl.ds(base + q * 16, 16)]
            d[q] = jnp.abs(pv - tv)
        a0 = a0 + d[0] + d[4]
        a1 = a1 + d[1] + d[5]
        a2 = a2 + d[2] + d[6]
        a3 = a3 + d[3] + d[7]
        return (a0, a1, a2, a3)

    accs = (zero, zero, zero, zero)
    for k in range(NCHUNK):
        pltpu.make_async_copy(out_hbm.at[pl.ds(0, EPC)],
                              pred2.at[pl.ds(k * EPC, EPC)], sems[k]).wait()
        accs = lax.fori_loop(k * RPC, (k + 1) * RPC, comp, accs, unroll=False)

    for q in range(4):
        accv[pl.ds(q * 16, 16)] = accs[q] * SCALE

    pltpu.sync_copy(accv, shared.at[pl.ds(sid * C, C)])
    plsc.subcore_barrier()

    @pl.when(sid == 0)
    def _():
        pltpu.sync_copy(shared, finv)

        def red(r, accs):
            a0, a1, a2, a3 = accs
            b = r * C
            a0 = a0 + finv[pl.ds(b, 16)]
            a1 = a1 + finv[pl.ds(b + 16, 16)]
            a2 = a2 + finv[pl.ds(b + 32, 16)]
            a3 = a3 + finv[pl.ds(b + 48, 16)]
            return (a0, a1, a2, a3)

        f = lax.fori_loop(0, NS, red, (zero, zero, zero, zero), unroll=False)
        for q in range(4):
            accv[pl.ds(q * 16, 16)] = f[q]
        pltpu.sync_copy(accv, res_hbm.at[cid])


_sc_call = pl.kernel(
    _body,
    out_type=jax.ShapeDtypeStruct((NC, C), jnp.float32),
    mesh=plsc.VectorSubcoreMesh(core_axis_name="c", subcore_axis_name="s",
                                num_cores=NC, num_subcores=NS),
    compiler_params=pltpu.CompilerParams(needs_layout_passes=False),
    scratch_types=[
        pltpu.VMEM((NPW,), jnp.int32),
        pltpu.VMEM((NPW,), jnp.int32),
        pltpu.VMEM((EPW,), jnp.int32),
        pltpu.VMEM((EPW,), jnp.float32),
        pltpu.VMEM((EPW,), jnp.float32),
        pltpu.VMEM((C,), jnp.float32),
        pltpu.VMEM((NS * C,), jnp.float32),
        pltpu.VMEM_SHARED((NS * C,), jnp.float32),
        pltpu.SemaphoreType.DMA,
        pltpu.SemaphoreType.DMA,
        pltpu.SemaphoreType.DMA,
        pltpu.SemaphoreType.DMA,
        pltpu.SemaphoreType.DMA,
        pltpu.SemaphoreType.DMA,
        pltpu.SemaphoreType.DMA,
        pltpu.SemaphoreType.DMA,
        pltpu.SemaphoreType.DMA,
        pltpu.SemaphoreType.DMA,
        pltpu.SemaphoreType.DMA,
        pltpu.SemaphoreType.DMA,
        pltpu.SemaphoreType.DMA,
        pltpu.SemaphoreType.DMA,
        pltpu.SemaphoreType.DMA,
        pltpu.SemaphoreType.DMA,
        pltpu.SemaphoreType.DMA,
    ],
)


def kernel(output, target, ind, batch):
    out_flat = output.reshape(-1)
    tgt_flat = target.reshape(-1)
    ind32 = ind.astype(jnp.int32)
    bat32 = batch.astype(jnp.int32)
    partials = _sc_call(out_flat, tgt_flat, ind32, bat32)
    return partials[0] + partials[1]


# speedup vs baseline: 1.0243x; 1.0243x over previous
import jax
import jax.numpy as jnp
from jax import lax
from jax.experimental import pallas as pl
from jax.experimental.pallas import tpu as pltpu
from jax.experimental.pallas import tpu_sc as plsc

B, C, H, W = 16, 64, 128, 128
HW = H * W
CHW = C * HW
N = 8192
NC, NS = 2, 16
NW = NC * NS
NPW = N // NW
EPW = NPW * C
ROWS = EPW // 128
SCALE = 1.0 / (N + 0.0001)


NCHUNK = 8
GPC = (NPW // 16) // NCHUNK
RPC = ROWS // NCHUNK
EPC = EPW // NCHUNK


def _body(out_hbm, tgt_hbm, ind_hbm, bat_hbm, res_hbm,
          indv, batv, idx2, pred2, tgtv, accv, finv, shared,
          sem_t, sem_g0, sem_g1, sem_g2, sem_g3,
          sem_g4, sem_g5, sem_g6, sem_g7,
          sem_g8, sem_g9, sem_ga, sem_gb,
          sem_gc, sem_gd, sem_ge, sem_gf):
    cid = lax.axis_index("c")
    sid = lax.axis_index("s")
    wid = cid * NS + sid
    n0 = wid * NPW
    sems = (sem_g0, sem_g1, sem_g2, sem_g3, sem_g4, sem_g5, sem_g6, sem_g7,
            sem_g8, sem_g9, sem_ga, sem_gb, sem_gc, sem_gd, sem_ge, sem_gf)

    tcopy = pltpu.async_copy(tgt_hbm.at[pl.ds(n0, NPW), :], tgtv, sem_t)
    icopy = pltpu.async_copy(ind_hbm.at[pl.ds(n0, NPW)], indv, sem_g0)
    bcopy = pltpu.async_copy(bat_hbm.at[pl.ds(n0, NPW)], batv, sem_g0)
    icopy.wait()
    bcopy.wait()

    iota = lax.iota(jnp.int32, 16)

    def build(g, carry):
        ind16 = indv[pl.ds(g * 16, 16)]
        bat16 = batv[pl.ds(g * 16, 16)]
        base16 = bat16 * CHW + ind16
        p0 = (g * 16 + iota) * C
        for c in range(C):
            plsc.store_scatter(idx2, [p0 + c], base16 + c * HW)
        return carry

    for k in range(NCHUNK):
        lax.fori_loop(k * GPC, (k + 1) * GPC, build, 0, unroll=False)
        pltpu.async_copy(out_hbm.at[idx2.at[pl.ds(k * EPC, EPC)]],
                         pred2.at[pl.ds(k * EPC, EPC)], sems[k])

    tcopy.wait()

    zero = jnp.zeros((16,), jnp.float32)

    def comp(j, accs):
        a0, a1, a2, a3 = accs
        base = j * 128
        d = [None] * 8
        for q in range(8):
            pv = pred2[pl.ds(base + q * 16, 16)]
            tv = tgtv[j * 2 + q // 4, pl.ds((q % 4) * 16, 16)]
            d[q] = jnp.abs(pv - tv)
        a0 = a0 + d[0] + d[4]
        a1 = a1 + d[1] + d[5]
        a2 = a2 + d[2] + d[6]
        a3 = a3 + d[3] + d[7]
        return (a0, a1, a2, a3)

    accs = (zero, zero, zero, zero)
    for k in range(NCHUNK):
        pltpu.make_async_copy(out_hbm.at[pl.ds(0, EPC)],
                              pred2.at[pl.ds(k * EPC, EPC)], sems[k]).wait()
        accs = lax.fori_loop(k * RPC, (k + 1) * RPC, comp, accs, unroll=False)

    for q in range(4):
        accv[pl.ds(q * 16, 16)] = accs[q] * SCALE

    pltpu.sync_copy(accv, shared.at[pl.ds(sid * C, C)])
    plsc.subcore_barrier()

    @pl.when(sid == 0)
    def _():
        pltpu.sync_copy(shared, finv)

        def red(r, accs):
            a0, a1, a2, a3 = accs
            b = r * C
            a0 = a0 + finv[pl.ds(b, 16)]
            a1 = a1 + finv[pl.ds(b + 16, 16)]
            a2 = a2 + finv[pl.ds(b + 32, 16)]
            a3 = a3 + finv[pl.ds(b + 48, 16)]
            return (a0, a1, a2, a3)

        f = lax.fori_loop(0, NS, red, (zero, zero, zero, zero), unroll=False)
        for q in range(4):
            accv[pl.ds(q * 16, 16)] = f[q]
        pltpu.sync_copy(accv, res_hbm.at[cid])


_sc_call = pl.kernel(
    _body,
    out_type=jax.ShapeDtypeStruct((NC, C), jnp.float32),
    mesh=plsc.VectorSubcoreMesh(core_axis_name="c", subcore_axis_name="s",
                                num_cores=NC, num_subcores=NS),
    compiler_params=pltpu.CompilerParams(needs_layout_passes=False),
    scratch_types=[
        pltpu.VMEM((NPW,), jnp.int32),
        pltpu.VMEM((NPW,), jnp.int32),
        pltpu.VMEM((EPW,), jnp.int32),
        pltpu.VMEM((EPW,), jnp.float32),
        pltpu.VMEM((NPW, C), jnp.float32),
        pltpu.VMEM((C,), jnp.float32),
        pltpu.VMEM((NS * C,), jnp.float32),
        pltpu.VMEM_SHARED((NS * C,), jnp.float32),
        pltpu.SemaphoreType.DMA,
        pltpu.SemaphoreType.DMA,
        pltpu.SemaphoreType.DMA,
        pltpu.SemaphoreType.DMA,
        pltpu.SemaphoreType.DMA,
        pltpu.SemaphoreType.DMA,
        pltpu.SemaphoreType.DMA,
        pltpu.SemaphoreType.DMA,
        pltpu.SemaphoreType.DMA,
        pltpu.SemaphoreType.DMA,
        pltpu.SemaphoreType.DMA,
        pltpu.SemaphoreType.DMA,
        pltpu.SemaphoreType.DMA,
        pltpu.SemaphoreType.DMA,
        pltpu.SemaphoreType.DMA,
        pltpu.SemaphoreType.DMA,
        pltpu.SemaphoreType.DMA,
    ],
)


def kernel(output, target, ind, batch):
    out_flat = output.reshape(-1)
    ind32 = ind.astype(jnp.int32)
    bat32 = batch.astype(jnp.int32)
    partials = _sc_call(out_flat, target, ind32, bat32)
    return partials[0] + partials[1]
